# baseline (speedup 1.0000x reference)
import jax
import jax.numpy as jnp
from jax.experimental import pallas as pl
from jax.experimental.pallas import tpu as pltpu


def _vq_block_kernel(zm_ref, emb_ref, zq_ref, idx_ref, loss_ref,
                     emb2_ref, embt_ref, e2_ref):
    k_dim = emb_ref.shape[0]
    t_blk = zm_ref.shape[2]
    half = t_blk // 8
    n_total = zm_ref.shape[1] * t_blk * pl.num_programs(0)

    @pl.when(pl.program_id(0) == 0)
    def _init():
        emb0 = emb_ref[...]
        emb2_ref[...] = -2.0 * emb0
        embt_ref[...] = emb0.T.astype(jnp.bfloat16)
        e2_ref[...] = jnp.sum(emb0 * emb0, axis=1, keepdims=True)
        loss_ref[...] = jnp.zeros((1, 1), jnp.float32)

    emb = emb2_ref[...]
    embt = embt_ref[...]
    e2 = e2_ref[...]
    iota_k = jax.lax.broadcasted_iota(jnp.int32, (k_dim, half), 0)

    def _half(zb):
        scores = jnp.dot(emb, zb, preferred_element_type=jnp.float32)
        dist = e2 + scores
        idx = jnp.argmin(dist, axis=0)
        onehot = (iota_k == idx[None, :]).astype(jnp.bfloat16)
        zq = jnp.dot(embt, onehot,
                     preferred_element_type=jnp.float32)
        resid = zb - zq
        part = jnp.sum(resid * resid, axis=(0, 1), keepdims=True)
        return zq, idx, part

    acc = None
    for q in range(8):
        sl = pl.ds(q * half, half)
        zq_q, idx_q, part_q = _half(zm_ref[0, :, sl])
        zq_ref[0, :, sl] = zq_q
        idx_ref[0, 0, sl] = idx_q
        acc = part_q if acc is None else acc + part_q
    loss_ref[...] += acc

    @pl.when(pl.program_id(0) == pl.num_programs(0) - 1)
    def _fin():
        loss_ref[...] = loss_ref[...] * (0.25 / n_total)


@jax.jit
def kernel(z, embedding):
    B, D, T = z.shape
    K = embedding.shape[0]

    grid = (B,)
    zq, idx3, loss_out = pl.pallas_call(
        _vq_block_kernel,
        grid=grid,
        in_specs=[
            pl.BlockSpec((1, D, T), lambda b: (b, 0, 0)),
            pl.BlockSpec((K, D), lambda b: (0, 0)),
        ],
        out_specs=[
            pl.BlockSpec((1, D, T), lambda b: (b, 0, 0)),
            pl.BlockSpec((1, 1, T), lambda b: (b, 0, 0)),
            pl.BlockSpec((1, 1), lambda b: (0, 0)),
        ],
        out_shape=[
            jax.ShapeDtypeStruct((B, D, T), jnp.float32),
            jax.ShapeDtypeStruct((B, 1, T), jnp.int32),
            jax.ShapeDtypeStruct((1, 1), jnp.float32),
        ],
        scratch_shapes=[
            pltpu.VMEM((K, D), jnp.float32),
            pltpu.VMEM((D, K), jnp.bfloat16),
            pltpu.VMEM((K, 1), jnp.float32),
        ],
        compiler_params=pltpu.CompilerParams(
            dimension_semantics=("arbitrary",),
        ),
    )(z, embedding)

    return zq, loss_out[0, 0], idx3.reshape(B, T)


# speedup vs baseline: 1.2099x; 1.2099x over previous
import jax
import jax.numpy as jnp
from jax.experimental import pallas as pl
from jax.experimental.pallas import tpu as pltpu


def _vq_block_kernel(zm_ref, emb_ref, zq_ref, idx_ref, loss_ref,
                     emb2_ref, embt_ref, e2_ref):
    k_dim = emb_ref.shape[0]
    t_blk = zm_ref.shape[2]
    half = t_blk // 2
    n_total = zm_ref.shape[1] * t_blk * pl.num_programs(0)

    @pl.when(pl.program_id(0) == 0)
    def _init():
        emb0 = emb_ref[...]
        emb2_ref[...] = -2.0 * emb0
        embt_ref[...] = emb0.T.astype(jnp.bfloat16)
        e2_ref[...] = jnp.sum(emb0 * emb0, axis=1, keepdims=True)
        loss_ref[...] = jnp.zeros((1, 1), jnp.float32)

    emb = emb2_ref[...]
    embt = embt_ref[...]
    e2 = e2_ref[...]
    iota_k = jax.lax.broadcasted_iota(jnp.int32, (k_dim, half), 0)

    def _half(zb):
        scores = jnp.dot(emb, zb, preferred_element_type=jnp.float32)
        dist = e2 + scores
        idx = jnp.argmin(dist, axis=0)
        onehot = (iota_k == idx[None, :]).astype(jnp.bfloat16)
        zq = jnp.dot(embt, onehot,
                     preferred_element_type=jnp.float32)
        resid = zb - zq
        part = jnp.sum(resid * resid, axis=(0, 1), keepdims=True)
        return zq, idx, part

    acc = None
    for q in range(2):
        sl = pl.ds(q * half, half)
        zq_q, idx_q, part_q = _half(zm_ref[0, :, sl])
        zq_ref[0, :, sl] = zq_q
        idx_ref[0, 0, sl] = idx_q
        acc = part_q if acc is None else acc + part_q
    loss_ref[...] += acc

    @pl.when(pl.program_id(0) == pl.num_programs(0) - 1)
    def _fin():
        loss_ref[...] = loss_ref[...] * (0.25 / n_total)


@jax.jit
def kernel(z, embedding):
    B, D, T = z.shape
    K = embedding.shape[0]

    t_blk = 512
    nt = T // t_blk
    grid = (B * nt,)
    zq, idx3, loss_out = pl.pallas_call(
        _vq_block_kernel,
        grid=grid,
        in_specs=[
            pl.BlockSpec((1, D, t_blk), lambda i: (i // nt, 0, i % nt)),
            pl.BlockSpec((K, D), lambda i: (0, 0)),
        ],
        out_specs=[
            pl.BlockSpec((1, D, t_blk), lambda i: (i // nt, 0, i % nt)),
            pl.BlockSpec((1, 1, t_blk), lambda i: (i, 0, 0)),
            pl.BlockSpec((1, 1), lambda i: (0, 0)),
        ],
        out_shape=[
            jax.ShapeDtypeStruct((B, D, T), jnp.float32),
            jax.ShapeDtypeStruct((B * nt, 1, t_blk), jnp.int32),
            jax.ShapeDtypeStruct((1, 1), jnp.float32),
        ],
        scratch_shapes=[
            pltpu.VMEM((K, D), jnp.float32),
            pltpu.VMEM((D, K), jnp.bfloat16),
            pltpu.VMEM((K, 1), jnp.float32),
        ],
        compiler_params=pltpu.CompilerParams(
            dimension_semantics=("arbitrary",),
        ),
    )(z, embedding)

    return zq, loss_out[0, 0], idx3.reshape(B, T)


# speedup vs baseline: 1.9405x; 1.6039x over previous
import jax
import jax.numpy as jnp
from jax.experimental import pallas as pl
from jax.experimental.pallas import tpu as pltpu


def _vq_block_kernel(zm_ref, emb_ref, zq_ref, idx_ref, loss_ref,
                     emb2_ref, embt_ref, e2_ref):
    k_dim = emb_ref.shape[0]
    t_blk = zm_ref.shape[2]
    half = t_blk // 4
    n_total = zm_ref.shape[0] * zm_ref.shape[1] * t_blk * pl.num_programs(0)

    @pl.when(pl.program_id(0) == 0)
    def _init():
        emb0 = emb_ref[...]
        emb2_ref[...] = -2.0 * emb0
        embt_ref[...] = emb0.T.astype(jnp.bfloat16)
        e2_ref[...] = jnp.sum(emb0 * emb0, axis=1, keepdims=True)
        loss_ref[...] = jnp.zeros((1, 1), jnp.float32)

    emb = emb2_ref[...]
    embt = embt_ref[...]
    e2 = e2_ref[...]
    iota_k = jax.lax.broadcasted_iota(jnp.int32, (k_dim, half), 0)

    def _half(zb):
        scores = jnp.dot(emb, zb, preferred_element_type=jnp.float32)
        dist = e2 + scores
        idx = jnp.argmin(dist, axis=0)
        onehot = (iota_k == idx[None, :]).astype(jnp.bfloat16)
        zq = jnp.dot(embt, onehot,
                     preferred_element_type=jnp.float32)
        resid = zb - zq
        part = jnp.sum(resid * resid, axis=(0, 1), keepdims=True)
        return zq, idx, part

    acc = None
    for bb in range(zm_ref.shape[0]):
        for q in range(4):
            sl = pl.ds(q * half, half)
            zq_q, idx_q, part_q = _half(zm_ref[bb, :, sl])
            zq_ref[bb, :, sl] = zq_q
            idx_ref[bb, 0, sl] = idx_q
            acc = part_q if acc is None else acc + part_q
    loss_ref[...] += acc

    @pl.when(pl.program_id(0) == pl.num_programs(0) - 1)
    def _fin():
        loss_ref[...] = loss_ref[...] * (0.25 / n_total)


@jax.jit
def kernel(z, embedding):
    B, D, T = z.shape
    K = embedding.shape[0]

    nb = 2
    grid = (B // nb,)
    zq, idx3, loss_out = pl.pallas_call(
        _vq_block_kernel,
        grid=grid,
        in_specs=[
            pl.BlockSpec((nb, D, T), lambda i: (i, 0, 0)),
            pl.BlockSpec((K, D), lambda i: (0, 0)),
        ],
        out_specs=[
            pl.BlockSpec((nb, D, T), lambda i: (i, 0, 0)),
            pl.BlockSpec((nb, 1, T), lambda i: (i, 0, 0)),
            pl.BlockSpec((1, 1), lambda i: (0, 0)),
        ],
        out_shape=[
            jax.ShapeDtypeStruct((B, D, T), jnp.float32),
            jax.ShapeDtypeStruct((B, 1, T), jnp.int32),
            jax.ShapeDtypeStruct((1, 1), jnp.float32),
        ],
        scratch_shapes=[
            pltpu.VMEM((K, D), jnp.float32),
            pltpu.VMEM((D, K), jnp.bfloat16),
            pltpu.VMEM((K, 1), jnp.float32),
        ],
        compiler_params=pltpu.CompilerParams(
            dimension_semantics=("arbitrary",),
        ),
    )(z, embedding)

    return zq, loss_out[0, 0], idx3.reshape(B, T)


# speedup vs baseline: 2.0788x; 1.0712x over previous
import jax
import jax.numpy as jnp
from jax.experimental import pallas as pl
from jax.experimental.pallas import tpu as pltpu


def _vq_block_kernel(zm_ref, emb_ref, zq_ref, idx_ref, loss_ref,
                     emb2_ref, embt_ref, e2_ref):
    k_dim = emb_ref.shape[0]
    t_blk = zm_ref.shape[2]
    half = t_blk // 4
    n_total = zm_ref.shape[0] * zm_ref.shape[1] * t_blk * pl.num_programs(0)

    @pl.when(pl.program_id(0) == 0)
    def _init():
        emb0 = emb_ref[...]
        emb2_ref[...] = -2.0 * emb0
        embt_ref[...] = emb0.T.astype(jnp.bfloat16)
        e2_ref[...] = jnp.sum(emb0 * emb0, axis=1, keepdims=True)
        loss_ref[...] = jnp.zeros((1, 1), jnp.float32)

    emb = emb2_ref[...]
    embt = embt_ref[...]
    e2 = e2_ref[...]
    iota_k = jax.lax.broadcasted_iota(jnp.int32, (k_dim, half), 0)

    def _half(zb):
        scores = jnp.dot(emb, zb, preferred_element_type=jnp.float32)
        dist = e2 + scores
        idx = jnp.argmin(dist, axis=0)
        onehot = (iota_k == idx[None, :]).astype(jnp.bfloat16)
        zq = jnp.dot(embt, onehot,
                     preferred_element_type=jnp.float32)
        resid = zb - zq
        part = jnp.sum(resid * resid, axis=(0, 1), keepdims=True)
        return zq, idx, part

    acc = None
    for bb in range(zm_ref.shape[0]):
        for q in range(4):
            sl = pl.ds(q * half, half)
            zq_q, idx_q, part_q = _half(zm_ref[bb, :, sl])
            zq_ref[bb, :, sl] = zq_q
            idx_ref[bb, 0, sl] = idx_q
            acc = part_q if acc is None else acc + part_q
    loss_ref[...] += acc

    @pl.when(pl.program_id(0) == pl.num_programs(0) - 1)
    def _fin():
        loss_ref[...] = loss_ref[...] * (0.25 / n_total)


@jax.jit
def kernel(z, embedding):
    B, D, T = z.shape
    K = embedding.shape[0]

    nb = 4
    grid = (B // nb,)
    zq, idx3, loss_out = pl.pallas_call(
        _vq_block_kernel,
        grid=grid,
        in_specs=[
            pl.BlockSpec((nb, D, T), lambda i: (i, 0, 0)),
            pl.BlockSpec((K, D), lambda i: (0, 0)),
        ],
        out_specs=[
            pl.BlockSpec((nb, D, T), lambda i: (i, 0, 0)),
            pl.BlockSpec((nb, 1, T), lambda i: (i, 0, 0)),
            pl.BlockSpec((1, 1), lambda i: (0, 0)),
        ],
        out_shape=[
            jax.ShapeDtypeStruct((B, D, T), jnp.float32),
            jax.ShapeDtypeStruct((B, 1, T), jnp.int32),
            jax.ShapeDtypeStruct((1, 1), jnp.float32),
        ],
        scratch_shapes=[
            pltpu.VMEM((K, D), jnp.float32),
            pltpu.VMEM((D, K), jnp.bfloat16),
            pltpu.VMEM((K, 1), jnp.float32),
        ],
        compiler_params=pltpu.CompilerParams(
            dimension_semantics=("arbitrary",),
        ),
    )(z, embedding)

    return zq, loss_out[0, 0], idx3.reshape(B, T)
